# baseline (speedup 1.0000x reference)
import functools

import jax
import jax.numpy as jnp
from jax import lax
from jax.experimental import pallas as pl
from jax.experimental.pallas import tpu as pltpu
from jax.experimental.pallas import tpu_sc as plsc

N = 10000
E = 320000
F = 128
H = 64
OUT = 64
B = 64
GH = 32
FH = 64

NC = 2
NS = 16
CHUNK = 128
NBUF = 4
NCH = -(-E // (NC * NS * CHUNK * NBUF)) * NBUF
EPAD = NC * NS * NCH * CHUNK
NPAD = -(-N // (NS * 8)) * NS * 8
ROWS_PER_TILE = NPAD // NS


def _edge_agg_body(y_hbm, src_hbm, dst_hbm, zeros_hbm, out_hbm,
                   src_v, dst_v, rows_v, acc_sh, sem):
    c = lax.axis_index("c")
    s = lax.axis_index("s")
    pltpu.sync_copy(zeros_hbm.at[pl.ds(s * ROWS_PER_TILE, ROWS_PER_TILE)],
                    acc_sh.at[pl.ds(s * ROWS_PER_TILE, ROWS_PER_TILE)])
    pltpu.sync_copy(src_hbm.at[c, s], src_v)
    pltpu.sync_copy(dst_hbm.at[c, s], dst_v)
    plsc.subcore_barrier()

    def body(j, carry):
        pltpu.async_copy(y_hbm.at[src_v.at[j]], rows_v, sem).wait()
        pltpu.sync_copy(rows_v, acc_sh.at[dst_v.at[j]], add=True)
        return carry

    lax.fori_loop(0, NCH, body, 0)
    plsc.subcore_barrier()
    pltpu.sync_copy(acc_sh.at[pl.ds(s * ROWS_PER_TILE, ROWS_PER_TILE)],
                    out_hbm.at[c, pl.ds(s * ROWS_PER_TILE, ROWS_PER_TILE)])


@functools.cache
def _make_edge_agg(width):
    return pl.kernel(
        _edge_agg_body,
        out_type=jax.ShapeDtypeStruct((NC, NPAD, width), jnp.float32),
        mesh=plsc.VectorSubcoreMesh(core_axis_name="c", subcore_axis_name="s",
                                    num_cores=NC, num_subcores=NS),
        scratch_types=[
            pltpu.VMEM((NCH, CHUNK), jnp.int32),
            pltpu.VMEM((NCH, CHUNK), jnp.int32),
            pltpu.VMEM((CHUNK, width), jnp.float32),
            pltpu.VMEM_SHARED((NPAD, width), jnp.float32),
            pltpu.SemaphoreType.DMA,
        ],
        compiler_params=pltpu.CompilerParams(use_tc_tiling_on_sc=False),
        name=f"edge_agg_{width}",
    )


def _edge_agg(y, src, dst, zeros):
    return _make_edge_agg(y.shape[-1])(y, src, dst, zeros)


def _prep_edges(edge_index):
    ei = edge_index.astype(jnp.int32)
    pad = EPAD - E
    src = jnp.concatenate([ei[0], jnp.zeros((pad,), jnp.int32)])
    dst = jnp.concatenate([ei[1], jnp.full((pad,), N, jnp.int32)])
    return (src.reshape(NC, NS, NCH, CHUNK), dst.reshape(NC, NS, NCH, CHUNK))


def _bn_relu(z, g, bt):
    mu = jnp.mean(z, axis=0, keepdims=True)
    var = jnp.mean((z - mu) * (z - mu), axis=0, keepdims=True)
    return jnp.maximum(g * (z - mu) * lax.rsqrt(var + 1e-5) + bt, 0.0)


def _conv_bn_relu(acc_ref, x_ref, wrel_ref, wroot_ref, b_ref, g_ref, bt_ref):
    acc = acc_ref[...]
    z = (jnp.dot(acc[0, :N] + acc[1, :N], wrel_ref[...],
                 preferred_element_type=jnp.float32)
         + jnp.dot(x_ref[...], wroot_ref[...],
                   preferred_element_type=jnp.float32)
         + b_ref[...])
    return _bn_relu(z, g_ref[...], bt_ref[...])


def _tc_mid_body(acc_ref, x_ref, wrel_ref, wroot_ref, b_ref, g_ref, bt_ref,
                 h_ref):
    h_ref[...] = _conv_bn_relu(acc_ref, x_ref, wrel_ref, wroot_ref, b_ref,
                               g_ref, bt_ref)


def _tc_mid(acc, x, wrel, wroot, b, g, bt):
    return pl.pallas_call(
        _tc_mid_body,
        out_shape=jax.ShapeDtypeStruct((N, wrel.shape[1]), jnp.float32),
    )(acc, x, wrel, wroot, b.reshape(1, -1), g.reshape(1, -1),
      bt.reshape(1, -1))


def _tc_pool_body(acc_ref, x_ref, wrel_ref, wroot_ref, b_ref, g_ref, bt_ref,
                  gw1_ref, gb1_ref, gw2_ref, gb2_ref, batch_ref, e_ref):
    h = _conv_bn_relu(acc_ref, x_ref, wrel_ref, wroot_ref, b_ref, g_ref,
                      bt_ref)
    t = jnp.maximum(jnp.dot(h, gw1_ref[...],
                            preferred_element_type=jnp.float32) + gb1_ref[...],
                    0.0)
    gate = jnp.maximum(jnp.dot(t, gw2_ref[...],
                               preferred_element_type=jnp.float32)
                       + gb2_ref[...], 0.0)
    oh = batch_ref[...] == lax.broadcasted_iota(jnp.int32, (1, B), 1)
    ohf = oh.astype(jnp.float32)
    m = jnp.max(jnp.where(oh, gate, -1e30), axis=0, keepdims=True)
    m_node = jnp.sum(ohf * m, axis=1, keepdims=True)
    e = jnp.exp(gate - m_node)
    seg = jnp.sum(ohf * e, axis=0, keepdims=True)
    s_node = jnp.sum(ohf * seg, axis=1, keepdims=True)
    a = e / (s_node + 1e-16)
    e_ref[...] = lax.dot_general(ohf, a * h, (((0,), (0,)), ((), ())),
                                 preferred_element_type=jnp.float32,
                                 precision=lax.Precision.HIGHEST)


def _tc_pool(acc, x, batch, p):
    return pl.pallas_call(
        _tc_pool_body,
        out_shape=jax.ShapeDtypeStruct((B, OUT), jnp.float32),
    )(acc, x, p['c3_Wrel'], p['c3_Wroot'], p['c3_b'].reshape(1, -1),
      p['n3_g'].reshape(1, -1), p['n3_b'].reshape(1, -1),
      p['g_W1'], p['g_b1'].reshape(1, -1), p['g_W2'], p['g_b2'].reshape(1, -1),
      batch.astype(jnp.int32).reshape(N, 1))


def _tc_final_body(e1_ref, e2_ref, w1_ref, b1_ref, w2_ref, b2_ref, o_ref):
    d = jnp.abs(e1_ref[...] - e2_ref[...])
    t = jnp.maximum(jnp.dot(d, w1_ref[...],
                            preferred_element_type=jnp.float32) + b1_ref[...],
                    0.0)
    o_ref[...] = (jnp.dot(t, w2_ref[...], preferred_element_type=jnp.float32)
                  + b2_ref[...])


def _tc_final(e1, e2, p):
    return pl.pallas_call(
        _tc_final_body,
        out_shape=jax.ShapeDtypeStruct((B, 1), jnp.float32),
    )(e1, e2, p['f_W1'], p['f_b1'].reshape(1, -1),
      p['f_W2'], p['f_b2'].reshape(1, -1))


def kernel(x1, edge_index1, batch1, x2, edge_index2, batch2, params):
    p = params
    zeros_f = jnp.zeros((NPAD, F), jnp.float32)
    zeros_h = jnp.zeros((NPAD, H), jnp.float32)
    s1, d1 = _prep_edges(edge_index1)
    s2, d2 = _prep_edges(edge_index2)

    aa = _edge_agg(x1, s1, d1, zeros_f)
    ab = _edge_agg(x2, s2, d2, zeros_f)
    ha = _tc_mid(aa, x1, p['c1_Wrel'], p['c1_Wroot'], p['c1_b'], p['n1_g'], p['n1_b'])
    hb = _tc_mid(ab, x2, p['c1_Wrel'], p['c1_Wroot'], p['c1_b'], p['n1_g'], p['n1_b'])
    aa = _edge_agg(ha, s1, d1, zeros_h)
    ab = _edge_agg(hb, s2, d2, zeros_h)
    ha = _tc_mid(aa, ha, p['c2_Wrel'], p['c2_Wroot'], p['c2_b'], p['n2_g'], p['n2_b'])
    hb = _tc_mid(ab, hb, p['c2_Wrel'], p['c2_Wroot'], p['c2_b'], p['n2_g'], p['n2_b'])
    aa = _edge_agg(ha, s1, d1, zeros_h)
    ab = _edge_agg(hb, s2, d2, zeros_h)
    e1 = _tc_pool(aa, ha, batch1, p)
    e2 = _tc_pool(ab, hb, batch2, p)
    return _tc_final(e1, e2, p)


# speedup vs baseline: 1.5137x; 1.5137x over previous
import functools

import jax
import jax.numpy as jnp
from jax import lax
from jax.experimental import pallas as pl
from jax.experimental.pallas import tpu as pltpu
from jax.experimental.pallas import tpu_sc as plsc

N = 10000
E = 320000
F = 128
H = 64
OUT = 64
B = 64
GH = 32
FH = 64

NC = 2
NS = 16
CHUNK = 128
NCH = -(-E // (NC * NS * CHUNK))
EPAD = NC * NS * NCH * CHUNK
NPAD = -(-N // (NS * 8)) * NS * 8
ROWS_PER_TILE = NPAD // NS


def _edge_agg_body(y_hbm, src_hbm, dst_hbm, zeros_hbm, out_hbm,
                   src_v, dst_v, rows_v, acc_sh, sem):
    c = lax.axis_index("c")
    s = lax.axis_index("s")
    pltpu.sync_copy(zeros_hbm.at[pl.ds(s * ROWS_PER_TILE, ROWS_PER_TILE)],
                    acc_sh.at[pl.ds(s * ROWS_PER_TILE, ROWS_PER_TILE)])
    pltpu.sync_copy(src_hbm.at[c, s], src_v)
    pltpu.sync_copy(dst_hbm.at[c, s], dst_v)
    plsc.subcore_barrier()

    def body(j, carry):
        pltpu.async_copy(y_hbm.at[src_v.at[j]], rows_v, sem).wait()
        pltpu.sync_copy(rows_v, acc_sh.at[dst_v.at[j]], add=True)
        return carry

    lax.fori_loop(0, NCH, body, 0)
    plsc.subcore_barrier()
    pltpu.sync_copy(acc_sh.at[pl.ds(s * ROWS_PER_TILE, ROWS_PER_TILE)],
                    out_hbm.at[c, pl.ds(s * ROWS_PER_TILE, ROWS_PER_TILE)])


@functools.cache
def _make_edge_agg(width):
    return pl.kernel(
        _edge_agg_body,
        out_type=jax.ShapeDtypeStruct((NC, NPAD, width), jnp.float32),
        mesh=plsc.VectorSubcoreMesh(core_axis_name="c", subcore_axis_name="s",
                                    num_cores=NC, num_subcores=NS),
        scratch_types=[
            pltpu.VMEM((NCH, CHUNK), jnp.int32),
            pltpu.VMEM((NCH, CHUNK), jnp.int32),
            pltpu.VMEM((CHUNK, width), jnp.float32),
            pltpu.VMEM_SHARED((NPAD, width), jnp.float32),
            pltpu.SemaphoreType.DMA,
        ],
        compiler_params=pltpu.CompilerParams(use_tc_tiling_on_sc=False),
        name=f"edge_agg_{width}",
    )


def _edge_agg(y, src, dst, zeros):
    return _make_edge_agg(y.shape[-1])(y, src, dst, zeros)


def _prep_edges(edge_index):
    ei = edge_index.astype(jnp.int32)
    pad = EPAD - E
    src = jnp.concatenate([ei[0], jnp.zeros((pad,), jnp.int32)])
    dst = jnp.concatenate(
        [ei[1], N + (jnp.arange(pad, dtype=jnp.int32) % (NPAD - N))])
    return (src.reshape(NC, NS, NCH, CHUNK), dst.reshape(NC, NS, NCH, CHUNK))


def _bn_relu(z, g, bt):
    mu = jnp.mean(z, axis=0, keepdims=True)
    var = jnp.mean((z - mu) * (z - mu), axis=0, keepdims=True)
    return jnp.maximum(g * (z - mu) * lax.rsqrt(var + 1e-5) + bt, 0.0)


def _conv_bn_relu(acc_ref, x_ref, wrel_ref, wroot_ref, b_ref, g_ref, bt_ref):
    acc = acc_ref[...]
    z = (jnp.dot(acc[0, :N] + acc[1, :N], wrel_ref[...],
                 preferred_element_type=jnp.float32)
         + jnp.dot(x_ref[...], wroot_ref[...],
                   preferred_element_type=jnp.float32)
         + b_ref[...])
    return _bn_relu(z, g_ref[...], bt_ref[...])


def _tc_mid_body(acc_ref, x_ref, wrel_ref, wroot_ref, b_ref, g_ref, bt_ref,
                 h_ref):
    h_ref[...] = _conv_bn_relu(acc_ref, x_ref, wrel_ref, wroot_ref, b_ref,
                               g_ref, bt_ref)


def _tc_mid(acc, x, wrel, wroot, b, g, bt):
    return pl.pallas_call(
        _tc_mid_body,
        out_shape=jax.ShapeDtypeStruct((N, wrel.shape[1]), jnp.float32),
    )(acc, x, wrel, wroot, b.reshape(1, -1), g.reshape(1, -1),
      bt.reshape(1, -1))


def _tc_pool_body(acc_ref, x_ref, wrel_ref, wroot_ref, b_ref, g_ref, bt_ref,
                  gw1_ref, gb1_ref, gw2_ref, gb2_ref, batch_ref, e_ref):
    h = _conv_bn_relu(acc_ref, x_ref, wrel_ref, wroot_ref, b_ref, g_ref,
                      bt_ref)
    t = jnp.maximum(jnp.dot(h, gw1_ref[...],
                            preferred_element_type=jnp.float32) + gb1_ref[...],
                    0.0)
    gate = jnp.maximum(jnp.dot(t, gw2_ref[...],
                               preferred_element_type=jnp.float32)
                       + gb2_ref[...], 0.0)
    oh = batch_ref[...] == lax.broadcasted_iota(jnp.int32, (1, B), 1)
    ohf = oh.astype(jnp.float32)
    m = jnp.max(jnp.where(oh, gate, -1e30), axis=0, keepdims=True)
    m_node = jnp.sum(ohf * m, axis=1, keepdims=True)
    e = jnp.exp(gate - m_node)
    seg = jnp.sum(ohf * e, axis=0, keepdims=True)
    s_node = jnp.sum(ohf * seg, axis=1, keepdims=True)
    a = e / (s_node + 1e-16)
    e_ref[...] = lax.dot_general(ohf, a * h, (((0,), (0,)), ((), ())),
                                 preferred_element_type=jnp.float32,
                                 precision=lax.Precision.HIGHEST)


def _tc_pool(acc, x, batch, p):
    return pl.pallas_call(
        _tc_pool_body,
        out_shape=jax.ShapeDtypeStruct((B, OUT), jnp.float32),
    )(acc, x, p['c3_Wrel'], p['c3_Wroot'], p['c3_b'].reshape(1, -1),
      p['n3_g'].reshape(1, -1), p['n3_b'].reshape(1, -1),
      p['g_W1'], p['g_b1'].reshape(1, -1), p['g_W2'], p['g_b2'].reshape(1, -1),
      batch.astype(jnp.int32).reshape(N, 1))


def _tc_final_body(e1_ref, e2_ref, w1_ref, b1_ref, w2_ref, b2_ref, o_ref):
    d = jnp.abs(e1_ref[...] - e2_ref[...])
    t = jnp.maximum(jnp.dot(d, w1_ref[...],
                            preferred_element_type=jnp.float32) + b1_ref[...],
                    0.0)
    o_ref[...] = (jnp.dot(t, w2_ref[...], preferred_element_type=jnp.float32)
                  + b2_ref[...])


def _tc_final(e1, e2, p):
    return pl.pallas_call(
        _tc_final_body,
        out_shape=jax.ShapeDtypeStruct((B, 1), jnp.float32),
    )(e1, e2, p['f_W1'], p['f_b1'].reshape(1, -1),
      p['f_W2'], p['f_b2'].reshape(1, -1))


def kernel(x1, edge_index1, batch1, x2, edge_index2, batch2, params):
    p = params
    zeros_f = jnp.zeros((NPAD, F), jnp.float32)
    zeros_h = jnp.zeros((NPAD, H), jnp.float32)
    s1, d1 = _prep_edges(edge_index1)
    s2, d2 = _prep_edges(edge_index2)

    aa = _edge_agg(x1, s1, d1, zeros_f)
    ab = _edge_agg(x2, s2, d2, zeros_f)
    ha = _tc_mid(aa, x1, p['c1_Wrel'], p['c1_Wroot'], p['c1_b'], p['n1_g'], p['n1_b'])
    hb = _tc_mid(ab, x2, p['c1_Wrel'], p['c1_Wroot'], p['c1_b'], p['n1_g'], p['n1_b'])
    aa = _edge_agg(ha, s1, d1, zeros_h)
    ab = _edge_agg(hb, s2, d2, zeros_h)
    ha = _tc_mid(aa, ha, p['c2_Wrel'], p['c2_Wroot'], p['c2_b'], p['n2_g'], p['n2_b'])
    hb = _tc_mid(ab, hb, p['c2_Wrel'], p['c2_Wroot'], p['c2_b'], p['n2_g'], p['n2_b'])
    aa = _edge_agg(ha, s1, d1, zeros_h)
    ab = _edge_agg(hb, s2, d2, zeros_h)
    e1 = _tc_pool(aa, ha, batch1, p)
    e2 = _tc_pool(ab, hb, batch2, p)
    return _tc_final(e1, e2, p)
